# baseline (speedup 1.0000x reference)
import functools

import jax
import jax.numpy as jnp
from jax import lax
from jax.experimental import pallas as pl
from jax.experimental.pallas import tpu as pltpu
from jax.experimental.pallas import tpu_sc as plsc

BATCH = 1024
HIDDEN = 64
VOCAB = 100000

_NC = 2
_NS = 16
_NW = _NC * _NS
_B_PER_W = BATCH // _NW


@functools.cache
def _make_sc_gather():
  mesh = plsc.VectorSubcoreMesh(
      core_axis_name="c", subcore_axis_name="s",
      num_cores=_NC, num_subcores=_NS)

  @functools.partial(
      pl.kernel,
      out_type=jax.ShapeDtypeStruct((BATCH, HIDDEN), jnp.float32),
      mesh=mesh,
      scratch_types=[
          pltpu.VMEM((_B_PER_W,), jnp.int32),
          pltpu.VMEM((_B_PER_W, HIDDEN), jnp.float32),
          pltpu.SemaphoreType.DMA,
      ],
      compiler_params=pltpu.CompilerParams(use_tc_tiling_on_sc=False),
  )
  def gather_kernel(idx_hbm, table_hbm, out_hbm, idx_v, rows_v, sem):
    wid = lax.axis_index("s") * _NC + lax.axis_index("c")
    base = wid * _B_PER_W
    pltpu.sync_copy(idx_hbm.at[pl.ds(base, _B_PER_W)], idx_v)
    pltpu.async_copy(table_hbm.at[idx_v], rows_v, sem).wait()
    pltpu.sync_copy(rows_v, out_hbm.at[pl.ds(base, _B_PER_W)])

  return gather_kernel


def _proj_body(x_ref, w_ref, b_ref, out_ref):
  out_ref[...] = jnp.dot(
      x_ref[...], w_ref[...], preferred_element_type=jnp.float32
  ) + b_ref[...]


_BV = 2048


@jax.jit
def kernel(input_ids, emb, W, b):
  x = _make_sc_gather()(input_ids.astype(jnp.int32), emb)

  grid = (pl.cdiv(VOCAB, _BV),)
  logits = pl.pallas_call(
      _proj_body,
      grid=grid,
      in_specs=[
          pl.BlockSpec((BATCH, HIDDEN), lambda j: (0, 0)),
          pl.BlockSpec((HIDDEN, _BV), lambda j: (0, j)),
          pl.BlockSpec((1, _BV), lambda j: (0, j)),
      ],
      out_specs=pl.BlockSpec((BATCH, _BV), lambda j: (0, j)),
      out_shape=jax.ShapeDtypeStruct((BATCH, VOCAB), jnp.float32),
  )(x, W, b.reshape(1, VOCAB))
  return logits
